# baseline (speedup 1.0000x reference)
import jax
import jax.numpy as jnp
from jax.experimental import pallas as pl
from jax.experimental.pallas import tpu as pltpu

_B, _C, _H, _W = 64, 768, 24, 24
_HW = _H * _W
_R, _L = (_C * _HW) // 1024, 1024


def _tile_body(t_ref, s_ref, emb_ref, tile_ref):
    del t_ref
    m = jax.nn.sigmoid(s_ref[0, 0] * emb_ref[0, 0, :])
    tile_ref[...] = jnp.broadcast_to(m[:, None], (_C, _HW))


def _mul_body(x_ref, tile_ref, out_ref, mask_ref):
    tile = tile_ref[...]
    out_ref[0] = x_ref[0] * tile
    mask_ref[0] = tile


def kernel(t, x, s, embedding):
    s2 = s.reshape(1, 1)
    t32 = t.astype(jnp.int32)

    tile = pl.pallas_call(
        _tile_body,
        grid_spec=pltpu.PrefetchScalarGridSpec(
            num_scalar_prefetch=1,
            grid=(1,),
            in_specs=[
                pl.BlockSpec((1, 1), lambda i, t_ref: (0, 0)),
                pl.BlockSpec((1, 1, _C), lambda i, t_ref: (t_ref[0], 0, 0)),
            ],
            out_specs=pl.BlockSpec((_C, _HW), lambda i, t_ref: (0, 0)),
        ),
        out_shape=jax.ShapeDtypeStruct((_C, _HW), jnp.float32),
    )(t32, s2, embedding.reshape(100, 1, _C))

    tile_flat = tile.reshape(_R, _L)
    x3 = x.reshape(_B, _R, _L)

    out, mask = pl.pallas_call(
        _mul_body,
        grid=(_B,),
        in_specs=[
            pl.BlockSpec((1, _R, _L), lambda b: (b, 0, 0)),
            pl.BlockSpec((_R, _L), lambda b: (0, 0)),
        ],
        out_specs=[
            pl.BlockSpec((1, _R, _L), lambda b: (b, 0, 0)),
            pl.BlockSpec((1, _R, _L), lambda b: (b, 0, 0)),
        ],
        out_shape=[
            jax.ShapeDtypeStruct((_B, _R, _L), jnp.float32),
            jax.ShapeDtypeStruct((_B, _R, _L), jnp.float32),
        ],
    )(x3, tile_flat)

    return out.reshape(x.shape), mask.reshape(x.shape)


# speedup vs baseline: 2.0016x; 2.0016x over previous
import jax
import jax.numpy as jnp
from jax.experimental import pallas as pl
from jax.experimental.pallas import tpu as pltpu

_B, _C, _H, _W = 64, 768, 24, 24
_HW = _H * _W
_BB = 2


def _mask_body(t_ref, s_ref, emb_ref, mask_ref):
    del t_ref
    m = jax.nn.sigmoid(s_ref[0, 0] * emb_ref[0, 0, :])
    mask_ref[...] = jnp.broadcast_to(m[None, :, None], (_BB, _C, _HW))


def _mul_body(t_ref, x_ref, s_ref, emb_ref, out_ref):
    del t_ref
    m = jax.nn.sigmoid(s_ref[0, 0] * emb_ref[0, 0, :])
    out_ref[...] = x_ref[...] * m[None, :, None]


def kernel(t, x, s, embedding):
    x3 = x.reshape(_B, _C, _HW)
    s2 = s.reshape(1, 1)
    t32 = t.astype(jnp.int32)
    emb3 = embedding.reshape(100, 1, _C)

    mask = pl.pallas_call(
        _mask_body,
        grid_spec=pltpu.PrefetchScalarGridSpec(
            num_scalar_prefetch=1,
            grid=(_B // _BB,),
            in_specs=[
                pl.BlockSpec((1, 1), lambda b, t_ref: (0, 0)),
                pl.BlockSpec((1, 1, _C), lambda b, t_ref: (t_ref[0], 0, 0)),
            ],
            out_specs=pl.BlockSpec((_BB, _C, _HW), lambda b, t_ref: (b, 0, 0)),
        ),
        out_shape=jax.ShapeDtypeStruct((_B, _C, _HW), jnp.float32),
    )(t32, s2, emb3)

    out = pl.pallas_call(
        _mul_body,
        grid_spec=pltpu.PrefetchScalarGridSpec(
            num_scalar_prefetch=1,
            grid=(_B // _BB,),
            in_specs=[
                pl.BlockSpec((_BB, _C, _HW), lambda b, t_ref: (b, 0, 0)),
                pl.BlockSpec((1, 1), lambda b, t_ref: (0, 0)),
                pl.BlockSpec((1, 1, _C), lambda b, t_ref: (t_ref[0], 0, 0)),
            ],
            out_specs=pl.BlockSpec((_BB, _C, _HW), lambda b, t_ref: (b, 0, 0)),
        ),
        out_shape=jax.ShapeDtypeStruct((_B, _C, _HW), jnp.float32),
    )(t32, x3, s2, emb3)

    return out.reshape(x.shape), mask.reshape(x.shape)
